# baseline (speedup 1.0000x reference)
import functools

import jax
import jax.numpy as jnp
from jax import lax
from jax.experimental import pallas as pl
from jax.experimental.pallas import tpu as pltpu
from jax.experimental.pallas import tpu_sc as plsc
from jax.experimental.layout import Layout, with_layout_constraint

BATCH = 16384
EMB = 16
WORD = 768
TROWS = 2_000_000
FIELD_OFFSET = 1_000_000

NC = 2
NS = 16
NW = NC * NS
BPW = BATCH // NW

_sc_mesh = plsc.VectorSubcoreMesh(core_axis_name="c", subcore_axis_name="s")


def _flat_emb_view(emb):
    x = emb.T.reshape(2, 8, TROWS // 128, 128)
    x = with_layout_constraint(
        x, Layout(major_to_minor=(0, 2, 1, 3), tiling=((8, 128),)))
    return x.transpose(0, 2, 1, 3).reshape(16 * TROWS)


@functools.partial(
    pl.kernel,
    out_type=jax.ShapeDtypeStruct((BATCH, 128), jnp.float32),
    mesh=_sc_mesh,
    compiler_params=pltpu.CompilerParams(use_tc_tiling_on_sc=False,
                                         needs_layout_passes=False),
    scratch_types=[
        pltpu.VMEM((2, BPW), jnp.int32),
        pltpu.VMEM((2, BPW * EMB), jnp.int32),
        pltpu.VMEM((2, BPW * EMB), jnp.float32),
        pltpu.VMEM((2, BPW, 1), jnp.float32),
        pltpu.VMEM((BPW, 128), jnp.float32),
        pltpu.SemaphoreType.DMA,
    ],
)
def _sc_gather(idx_hbm, emb_flat, lin_raw, out_hbm,
               idx_v, wv, rows, linv, out_v, sem):
    wid = lax.axis_index("s") * NC + lax.axis_index("c")
    for f in range(2):
        pltpu.sync_copy(idx_hbm.at[f, wid], idx_v.at[f])

    cps = [pltpu.async_copy(lin_raw.at[idx_v.at[f]], linv.at[f], sem)
           for f in range(2)]

    iota16 = lax.iota(jnp.int32, 16)
    epat = [(e // 8) * (8 * TROWS) + (e % 8) * 128 for e in range(EMB)]

    def wbody(g, c):
        posbase = iota16 * EMB + g * (16 * EMB)
        for f in range(2):
            iv = idx_v[f, pl.ds(g * 16, 16)]
            base = (iv // 128) * 1024 + iv % 128
            for e in range(EMB):
                plsc.store_scatter(wv.at[f], [posbase + e], base + epat[e])
        return c

    lax.fori_loop(0, BPW // 16, wbody, 0)

    for f in range(2):
        cps.append(pltpu.async_copy(emb_flat.at[wv.at[f]], rows.at[f], sem))
    for c in cps:
        c.wait()

    def body(k, c):
        e1 = rows[0, pl.ds(k * EMB, EMB)]
        e2 = rows[1, pl.ds(k * EMB, EMB)]
        out_v[k, pl.ds(0, EMB)] = e1 + e2
        out_v[k, pl.ds(EMB, EMB)] = e1 * e1 + e2 * e2
        return c

    lax.fori_loop(0, BPW, body, 0)

    col32 = jnp.full((16,), 2 * EMB, jnp.int32)
    zero16 = jnp.zeros((16,), jnp.int32)
    for g in range(BPW // 16):
        rws = iota16 + g * 16
        lsum = (plsc.load_gather(linv.at[0], [rws, zero16])
                + plsc.load_gather(linv.at[1], [rws, zero16]))
        plsc.store_scatter(out_v, [rws, col32], lsum)

    pltpu.sync_copy(out_v, out_hbm.at[pl.ds(wid * BPW, BPW)])


BB = 1024
BB2 = 2048


def _tc_text_body(u_ref, v_ref, w_ref, bt_ref, out_ref):
    w = w_ref[...]
    dn = (((1,), (1,)), ((), ()))
    t1 = lax.dot_general(u_ref[...], w, dn,
                         preferred_element_type=jnp.float32) + bt_ref[...]
    t2 = lax.dot_general(v_ref[...], w, dn,
                         preferred_element_type=jnp.float32) + bt_ref[...]
    out_ref[:, 0:EMB] = t1 + t2
    out_ref[:, EMB:2 * EMB] = t1 * t1 + t2 * t2


_tc_text = pl.pallas_call(
    _tc_text_body,
    grid=(BATCH // BB,),
    in_specs=[
        pl.BlockSpec((BB, WORD), lambda i: (i, 0)),
        pl.BlockSpec((BB, WORD), lambda i: (i, 0)),
        pl.BlockSpec((EMB, WORD), lambda i: (0, 0)),
        pl.BlockSpec((1, EMB), lambda i: (0, 0)),
    ],
    out_specs=pl.BlockSpec((BB, 128), lambda i: (i, 0)),
    out_shape=jax.ShapeDtypeStruct((BATCH, 128), jnp.float32),
)


def _tc_combine_body(sc_ref, tc_ref, bias_ref, out_ref):
    s = sc_ref[:, 0:EMB] + tc_ref[:, 0:EMB]
    q = sc_ref[:, EMB:2 * EMB] + tc_ref[:, EMB:2 * EMB]
    second = 0.5 * jnp.sum(s * s - q, axis=1, keepdims=True)
    out_ref[...] = second + sc_ref[:, 2 * EMB:2 * EMB + 1] + bias_ref[...]


_tc_combine = pl.pallas_call(
    _tc_combine_body,
    grid=(BATCH // BB2,),
    in_specs=[
        pl.BlockSpec((BB2, 128), lambda i: (i, 0)),
        pl.BlockSpec((BB2, 128), lambda i: (i, 0)),
        pl.BlockSpec((1, 1), lambda i: (0, 0)),
    ],
    out_specs=pl.BlockSpec((BB2, 1), lambda i: (i, 0)),
    out_shape=jax.ShapeDtypeStruct((BATCH, 1), jnp.float32),
)


def kernel(user_book_vector, user_text_vector, book_text_vector,
           emb_table, lin_table, lin_bias, W_text, b_text):
    offs = jnp.array([[0], [FIELD_OFFSET]], jnp.int32)
    idxs = (user_book_vector.T + offs).reshape(2, NW, BPW)

    emb_flat = _flat_emb_view(emb_table)

    sc_fat = _sc_gather(idxs, emb_flat, lin_table)
    tc_fat = _tc_text(user_text_vector, book_text_vector, W_text,
                      b_text.reshape(1, EMB))
    out = _tc_combine(sc_fat, tc_fat, lin_bias.reshape(1, 1))
    return out.reshape(BATCH)


# speedup vs baseline: 19.3602x; 19.3602x over previous
import functools

import jax
import jax.numpy as jnp
from jax import lax
from jax.experimental import pallas as pl
from jax.experimental.pallas import tpu as pltpu
from jax.experimental.pallas import tpu_sc as plsc
from jax.experimental.layout import Layout, with_layout_constraint

BATCH = 16384
EMB = 16
WORD = 768
TROWS = 2_000_000
FIELD_OFFSET = 1_000_000

NC = 2
NS = 16
NW = NC * NS
BPW = BATCH // NW

_sc_mesh = plsc.VectorSubcoreMesh(core_axis_name="c", subcore_axis_name="s")


def _flat_emb_view(emb):
    x = emb.T.reshape(2, 8, TROWS // 128, 128)
    x = with_layout_constraint(
        x, Layout(major_to_minor=(0, 2, 1, 3), tiling=((8, 128),)))
    return x.transpose(0, 2, 1, 3).reshape(16 * TROWS)


@functools.partial(
    pl.kernel,
    out_type=jax.ShapeDtypeStruct((BATCH, 128), jnp.float32),
    mesh=_sc_mesh,
    compiler_params=pltpu.CompilerParams(use_tc_tiling_on_sc=False,
                                         needs_layout_passes=False),
    scratch_types=[
        pltpu.VMEM((2, BPW), jnp.int32),
        pltpu.VMEM((2, BPW * EMB), jnp.int32),
        pltpu.VMEM((2, BPW * EMB), jnp.float32),
        pltpu.VMEM((2, BPW), jnp.float32),
        pltpu.VMEM((BPW, 128), jnp.float32),
        pltpu.SemaphoreType.DMA,
    ],
)
def _sc_gather(idx_hbm, emb_flat, lin_flat, out_hbm,
               idx_v, wv, rows, linv, out_v, sem):
    wid = lax.axis_index("s") * NC + lax.axis_index("c")
    for f in range(2):
        pltpu.sync_copy(idx_hbm.at[f, wid], idx_v.at[f])

    cps = [pltpu.async_copy(lin_flat.at[0].at[idx_v.at[f]], linv.at[f], sem)
           for f in range(2)]

    iota16 = lax.iota(jnp.int32, 16)
    epat = [(e // 8) * (8 * TROWS) + (e % 8) * 128 for e in range(EMB)]

    def wbody(g, c):
        posbase = iota16 * EMB + g * (16 * EMB)
        for f in range(2):
            iv = idx_v[f, pl.ds(g * 16, 16)]
            base = (iv // 128) * 1024 + iv % 128
            for e in range(EMB):
                plsc.store_scatter(wv.at[f], [posbase + e], base + epat[e])
        return c

    lax.fori_loop(0, BPW // 16, wbody, 0)

    for f in range(2):
        cps.append(pltpu.async_copy(emb_flat.at[wv.at[f]], rows.at[f], sem))
    for c in cps:
        c.wait()

    def body(k, c):
        e1 = rows[0, pl.ds(k * EMB, EMB)]
        e2 = rows[1, pl.ds(k * EMB, EMB)]
        out_v[k, pl.ds(0, EMB)] = e1 + e2
        out_v[k, pl.ds(EMB, EMB)] = e1 * e1 + e2 * e2
        return c

    lax.fori_loop(0, BPW, body, 0)

    col32 = jnp.full((16,), 2 * EMB, jnp.int32)
    for g in range(BPW // 16):
        lsum = linv[0, pl.ds(g * 16, 16)] + linv[1, pl.ds(g * 16, 16)]
        plsc.store_scatter(out_v, [iota16 + g * 16, col32], lsum)

    pltpu.sync_copy(out_v, out_hbm.at[pl.ds(wid * BPW, BPW)])


BB = 2048
BB2 = 2048


def _tc_text_body(u_ref, v_ref, w_ref, bt_ref, out_ref):
    w = w_ref[...]
    dn = (((1,), (1,)), ((), ()))
    t1 = lax.dot_general(u_ref[...], w, dn,
                         preferred_element_type=jnp.float32) + bt_ref[...]
    t2 = lax.dot_general(v_ref[...], w, dn,
                         preferred_element_type=jnp.float32) + bt_ref[...]
    out_ref[:, 0:EMB] = t1 + t2
    out_ref[:, EMB:2 * EMB] = t1 * t1 + t2 * t2


_tc_text = pl.pallas_call(
    _tc_text_body,
    grid=(BATCH // BB,),
    in_specs=[
        pl.BlockSpec((BB, WORD), lambda i: (i, 0)),
        pl.BlockSpec((BB, WORD), lambda i: (i, 0)),
        pl.BlockSpec((EMB, WORD), lambda i: (0, 0)),
        pl.BlockSpec((1, EMB), lambda i: (0, 0)),
    ],
    out_specs=pl.BlockSpec((BB, 128), lambda i: (i, 0)),
    out_shape=jax.ShapeDtypeStruct((BATCH, 128), jnp.float32),
)


def _tc_combine_body(sc_ref, tc_ref, bias_ref, out_ref):
    s = sc_ref[:, 0:EMB] + tc_ref[:, 0:EMB]
    q = sc_ref[:, EMB:2 * EMB] + tc_ref[:, EMB:2 * EMB]
    second = 0.5 * jnp.sum(s * s - q, axis=1, keepdims=True)
    out_ref[...] = second + sc_ref[:, 2 * EMB:2 * EMB + 1] + bias_ref[...]


_tc_combine = pl.pallas_call(
    _tc_combine_body,
    grid=(BATCH // BB2,),
    in_specs=[
        pl.BlockSpec((BB2, 128), lambda i: (i, 0)),
        pl.BlockSpec((BB2, 128), lambda i: (i, 0)),
        pl.BlockSpec((1, 1), lambda i: (0, 0)),
    ],
    out_specs=pl.BlockSpec((BB2, 1), lambda i: (i, 0)),
    out_shape=jax.ShapeDtypeStruct((BATCH, 1), jnp.float32),
)


def kernel(user_book_vector, user_text_vector, book_text_vector,
           emb_table, lin_table, lin_bias, W_text, b_text):
    offs = jnp.array([[0], [FIELD_OFFSET]], jnp.int32)
    idxs = (user_book_vector.T + offs).reshape(2, NW, BPW)

    emb_flat = _flat_emb_view(emb_table)

    sc_fat = _sc_gather(idxs, emb_flat, lin_table.T)
    tc_fat = _tc_text(user_text_vector, book_text_vector, W_text,
                      b_text.reshape(1, EMB))
    out = _tc_combine(sc_fat, tc_fat, lin_bias.reshape(1, 1))
    return out.reshape(BATCH)
